# baseline (speedup 1.0000x reference)
import jax
import jax.numpy as jnp
from jax.experimental import pallas as pl

NUM_NODES = 12
NUM_EDGES = 36
NODE_DIM = 128
TB = 256


def _fused_body(ew_ref, w_ref, b_ref, x_ref, out_ref, wts_ref):
    wts = jax.nn.softplus(ew_ref[:, :])
    wts_ref[:, :] = wts
    wmat = w_ref[:, :]
    bias = b_ref[:, :]
    uacc = [None] * 6
    vacc = [None] * 6
    for i in range(6):
        for j in range(6):
            e = 6 * i + j
            wf = x_ref[:, e * 128:(e + 1) * 128] * wts[0:1, e:e + 1]
            uacc[i] = wf if uacc[i] is None else uacc[i] + wf
            vacc[j] = wf if vacc[j] is None else vacc[j] + wf
    for n in range(NUM_NODES):
        acc = uacc[n] if n < 6 else vacc[n - 6]
        y = jax.lax.dot_general(acc, wmat, (((1,), (1,)), ((), ())),
                                preferred_element_type=jnp.float32)
        out_ref[:, n * 128:(n + 1) * 128] = jnp.maximum(y + bias, 0.0)


def kernel(edge_feats, edge_weights, W, b):
    B, E, D = edge_feats.shape
    xf = edge_feats.reshape(B, E * D)
    ew2 = edge_weights.reshape(1, E)
    b2 = b.reshape(1, D)
    grid = (B // TB,)
    nodes, wts = pl.pallas_call(
        _fused_body,
        grid=grid,
        in_specs=[
            pl.BlockSpec((1, E), lambda i: (0, 0)),
            pl.BlockSpec((D, D), lambda i: (0, 0)),
            pl.BlockSpec((1, D), lambda i: (0, 0)),
            pl.BlockSpec((TB, E * D), lambda i: (i, 0)),
        ],
        out_specs=[
            pl.BlockSpec((TB, NUM_NODES * D), lambda i: (i, 0)),
            pl.BlockSpec((1, E), lambda i: (0, 0)),
        ],
        out_shape=[
            jax.ShapeDtypeStruct((B, NUM_NODES * D), edge_feats.dtype),
            jax.ShapeDtypeStruct((1, E), edge_weights.dtype),
        ],
    )(ew2, W, b2, xf)
    return (nodes.reshape(B, NUM_NODES, D), wts.reshape(E))
